# Initial kernel scaffold; baseline (speedup 1.0000x reference)
import jax
import jax.numpy as jnp
from jax.experimental import pallas as pl


def kernel(seq_len, table):
    raise NotImplementedError("write your pallas kernel here")


# speedup vs baseline: 2.5992x; 2.5992x over previous
import functools

import jax
import jax.numpy as jnp
from jax import lax
from jax.experimental import pallas as pl
from jax.experimental.pallas import tpu as pltpu
from jax.experimental.pallas import tpu_sc as plsc

D_MODEL = 256
MAX_LEN = 2048
SEQ_LEN = 512
WIN_START = MAX_LEN - SEQ_LEN
WIN_ROWS = 2 * SEQ_LEN
ROW_ELEMS = SEQ_LEN * D_MODEL

NUM_CORES = 2
NUM_SUBCORES = 16
NUM_WORKERS = NUM_CORES * NUM_SUBCORES
ROWS_PER_WORKER = SEQ_LEN // NUM_WORKERS


def _body(table_hbm, out_hbm, win, sem):
    cid = lax.axis_index("c")
    sid = lax.axis_index("s")

    @pl.when(sid == 0)
    def _load():
        pltpu.sync_copy(
            table_hbm.at[pl.ds(WIN_START * D_MODEL, WIN_ROWS * D_MODEL)], win)

    plsc.subcore_barrier()

    wid = sid * NUM_CORES + cid
    base = wid * ROWS_PER_WORKER

    def step(k, carry):
        i = base + k
        src = pl.multiple_of(((SEQ_LEN - 1) - i) * D_MODEL, D_MODEL)
        dst = pl.multiple_of(i * ROW_ELEMS, ROW_ELEMS)
        pltpu.sync_copy(win.at[pl.ds(src, ROW_ELEMS)],
                        out_hbm.at[pl.ds(dst, ROW_ELEMS)])
        return carry

    lax.fori_loop(0, ROWS_PER_WORKER, step, 0)


def kernel(seq_len, table):
    del seq_len
    mesh = plsc.VectorSubcoreMesh(core_axis_name="c", subcore_axis_name="s")
    run = functools.partial(
        pl.kernel,
        mesh=mesh,
        out_type=jax.ShapeDtypeStruct((SEQ_LEN * SEQ_LEN * D_MODEL,), jnp.float32),
        scratch_types=[
            pltpu.VMEM_SHARED((WIN_ROWS * D_MODEL,), jnp.float32),
            pltpu.SemaphoreType.DMA,
        ],
    )(_body)
    flat = run(table.reshape(-1))
    return flat.reshape(SEQ_LEN, SEQ_LEN, D_MODEL)
